# baseline (speedup 1.0000x reference)
import functools

import jax
import jax.numpy as jnp
from jax import lax
from jax.experimental import pallas as pl
from jax.experimental.pallas import tpu as pltpu
from jax.experimental.pallas import tpu_sc as plsc

NUM_OBJ_CLS = 151
NUM_REL_CLS = 51
NUM_OBJS = 4096
NUM_RELS = 65536
TPAD = 128
DPAD = 64

NC = 2
NS = 16
L = 16
NW = NC * NS
B_PER_W = NUM_RELS // NW
CHUNK = 128
N_CHUNKS = B_PER_W // CHUNK
OH_PER_W = NUM_OBJS // NW
OH_WORDS = OH_PER_W * NUM_OBJ_CLS


def _idx_oh_body(labels_hbm, pairs_hbm, oh_hbm, idx_hbm,
                 labels_v, pairs_v, idx_v, oh_v):
    wid = lax.axis_index("s") * NC + lax.axis_index("c")
    base = wid * B_PER_W

    pltpu.sync_copy(labels_hbm, labels_v)
    pltpu.sync_copy(pairs_hbm.at[pl.ds(2 * base, 2 * B_PER_W)], pairs_v)

    lane = lax.broadcasted_iota(jnp.int32, (L,), 0)
    lane2 = 2 * lane
    zero16 = jnp.zeros((L,), jnp.float32)
    one16 = jnp.ones((L,), jnp.float32)

    def zstep(i, carry):
        oh_v[pl.ds(i * L, L)] = zero16
        return carry
    lax.fori_loop(0, OH_WORDS // L, zstep, 0)

    obase = wid * OH_PER_W
    for j in range(OH_PER_W // L):
        r16 = j * L + lane
        lbl = plsc.load_gather(labels_v, [obase + r16])
        plsc.store_scatter(oh_v, [r16 * NUM_OBJ_CLS + lbl], one16)
    pltpu.sync_copy(oh_v, oh_hbm.at[pl.ds(wid * OH_WORDS, OH_WORDS)])

    def step(i, carry):
        pos = i * (2 * L) + lane2
        h = plsc.load_gather(pairs_v, [pos])
        t = plsc.load_gather(pairs_v, [pos + 1])
        hl = plsc.load_gather(labels_v, [h])
        tl = plsc.load_gather(labels_v, [t])
        idx_v[pl.ds(i * L, L)] = hl * NUM_OBJ_CLS + tl
        return carry
    lax.fori_loop(0, B_PER_W // L, step, 0)
    pltpu.sync_copy(idx_v, idx_hbm.at[pl.ds(base, B_PER_W)])


_idx_oh = pl.kernel(
    _idx_oh_body,
    out_type=(
        jax.ShapeDtypeStruct((NUM_OBJS * NUM_OBJ_CLS,), jnp.float32),
        jax.ShapeDtypeStruct((NUM_RELS,), jnp.int32),
    ),
    mesh=plsc.VectorSubcoreMesh(
        core_axis_name="c", subcore_axis_name="s",
        num_cores=NC, num_subcores=NS),
    scratch_types=[
        pltpu.VMEM((NUM_OBJS,), jnp.int32),
        pltpu.VMEM((2 * B_PER_W,), jnp.int32),
        pltpu.VMEM((B_PER_W,), jnp.int32),
        pltpu.VMEM((OH_WORDS,), jnp.float32),
    ],
    compiler_params=pltpu.CompilerParams(
        needs_layout_passes=False, use_tc_tiling_on_sc=False),
)


def _rel_body(idx_hbm, table_hbm, out_hbm,
              idx_v, rows_a, rows_b, sem_g, sem_wa, sem_wb):
    wid = lax.axis_index("s") * NC + lax.axis_index("c")
    base = wid * B_PER_W

    pltpu.sync_copy(idx_hbm.at[pl.ds(base, B_PER_W)], idx_v)

    rows = [rows_a, rows_b]
    sems = [sem_wa, sem_wb]
    wb = [None, None]
    for k in range(N_CHUNKS):
        b = k % 2
        if wb[b] is not None:
            wb[b].wait()
        pltpu.async_copy(
            table_hbm.at[idx_v.at[pl.ds(k * CHUNK, CHUNK)]],
            rows[b], sem_g).wait()
        wb[b] = pltpu.async_copy(
            rows[b].at[:, pl.ds(0, DPAD)],
            out_hbm.at[pl.ds(base + k * CHUNK, CHUNK)], sems[b])
    wb[0].wait()
    wb[1].wait()


_rel_gather = pl.kernel(
    _rel_body,
    out_type=jax.ShapeDtypeStruct((NUM_RELS, DPAD), jnp.float32),
    mesh=plsc.VectorSubcoreMesh(
        core_axis_name="c", subcore_axis_name="s",
        num_cores=NC, num_subcores=NS),
    scratch_types=[
        pltpu.VMEM((B_PER_W,), jnp.int32),
        pltpu.VMEM((CHUNK, TPAD), jnp.float32),
        pltpu.VMEM((CHUNK, TPAD), jnp.float32),
        pltpu.SemaphoreType.DMA,
        pltpu.SemaphoreType.DMA,
        pltpu.SemaphoreType.DMA,
    ],
    compiler_params=pltpu.CompilerParams(
        needs_layout_passes=False, use_tc_tiling_on_sc=False),
)


@jax.jit
def kernel(obj_labels, rel_pair_idxs, prior_table):
    labels = obj_labels.astype(jnp.int32)
    pairs = rel_pair_idxs.astype(jnp.int32).reshape(2 * NUM_RELS)
    table128 = lax.pad(
        prior_table, jnp.float32(0.0),
        ((0, 0, 0), (0, 0, 0), (0, TPAD - NUM_REL_CLS, 0)),
    ).reshape(NUM_OBJ_CLS * NUM_OBJ_CLS, TPAD)
    oh_flat, idx = _idx_oh(labels, pairs)
    rel64 = _rel_gather(idx, table128)
    return (oh_flat.reshape(NUM_OBJS, NUM_OBJ_CLS), rel64[:, :NUM_REL_CLS])


# speedup vs baseline: 1.4506x; 1.4506x over previous
import functools

import jax
import jax.numpy as jnp
from jax import lax
from jax.experimental import pallas as pl
from jax.experimental.pallas import tpu as pltpu
from jax.experimental.pallas import tpu_sc as plsc

NUM_OBJ_CLS = 151
NUM_REL_CLS = 51
NUM_OBJS = 4096
NUM_RELS = 65536
TPAD = 64
DPAD = 64

NC = 2
NS = 16
L = 16
NW = NC * NS
B_PER_W = NUM_RELS // NW
CHUNK = 128
N_CHUNKS = B_PER_W // CHUNK
OH_PER_W = NUM_OBJS // NW
OH_WORDS = OH_PER_W * NUM_OBJ_CLS


def _idx_oh_body(labels_hbm, heads_hbm, tails_hbm, oh_hbm, idx_hbm,
                 labels_v, heads_v, tails_v, idx_v, oh_v):
    wid = lax.axis_index("s") * NC + lax.axis_index("c")
    base = wid * B_PER_W

    pltpu.sync_copy(labels_hbm, labels_v)
    pltpu.sync_copy(heads_hbm.at[pl.ds(base, B_PER_W)], heads_v)
    pltpu.sync_copy(tails_hbm.at[pl.ds(base, B_PER_W)], tails_v)

    lane = lax.broadcasted_iota(jnp.int32, (L,), 0)
    zero16 = jnp.zeros((L,), jnp.float32)
    one16 = jnp.ones((L,), jnp.float32)

    def zstep(i, carry):
        oh_v[pl.ds(i * L, L)] = zero16
        return carry
    lax.fori_loop(0, OH_WORDS // L, zstep, 0)

    obase = wid * OH_PER_W
    for j in range(OH_PER_W // L):
        r16 = j * L + lane
        lbl = plsc.load_gather(labels_v, [obase + r16])
        plsc.store_scatter(oh_v, [r16 * NUM_OBJ_CLS + lbl], one16)
    pltpu.sync_copy(oh_v, oh_hbm.at[pl.ds(wid * OH_WORDS, OH_WORDS)])

    def step(i, carry):
        pos = i * L + lane
        h = plsc.load_gather(heads_v, [pos])
        t = plsc.load_gather(tails_v, [pos])
        hl = plsc.load_gather(labels_v, [h])
        tl = plsc.load_gather(labels_v, [t])
        idx_v[pl.ds(i * L, L)] = hl * NUM_OBJ_CLS + tl
        return carry
    lax.fori_loop(0, B_PER_W // L, step, 0)
    pltpu.sync_copy(idx_v, idx_hbm.at[pl.ds(base, B_PER_W)])


_idx_oh = pl.kernel(
    _idx_oh_body,
    out_type=(
        jax.ShapeDtypeStruct((NUM_OBJS * NUM_OBJ_CLS,), jnp.float32),
        jax.ShapeDtypeStruct((NUM_RELS,), jnp.int32),
    ),
    mesh=plsc.VectorSubcoreMesh(
        core_axis_name="c", subcore_axis_name="s",
        num_cores=NC, num_subcores=NS),
    scratch_types=[
        pltpu.VMEM((NUM_OBJS,), jnp.int32),
        pltpu.VMEM((B_PER_W,), jnp.int32),
        pltpu.VMEM((B_PER_W,), jnp.int32),
        pltpu.VMEM((B_PER_W,), jnp.int32),
        pltpu.VMEM((OH_WORDS,), jnp.float32),
    ],
    compiler_params=pltpu.CompilerParams(
        needs_layout_passes=False, use_tc_tiling_on_sc=False),
)


def _rel_body(idx_hbm, table_hbm, out_hbm,
              idx_v, rows_a, rows_b, sem_g, sem_wa, sem_wb):
    wid = lax.axis_index("s") * NC + lax.axis_index("c")
    base = wid * B_PER_W

    pltpu.sync_copy(idx_hbm.at[pl.ds(base, B_PER_W)], idx_v)

    rows = [rows_a, rows_b]
    sems = [sem_wa, sem_wb]
    wb = [None, None]
    for k in range(N_CHUNKS):
        b = k % 2
        if wb[b] is not None:
            wb[b].wait()
        pltpu.async_copy(
            table_hbm.at[idx_v.at[pl.ds(k * CHUNK, CHUNK)]],
            rows[b], sem_g).wait()
        wb[b] = pltpu.async_copy(
            rows[b], out_hbm.at[pl.ds(base + k * CHUNK, CHUNK)], sems[b])
    wb[0].wait()
    wb[1].wait()


_rel_gather = pl.kernel(
    _rel_body,
    out_type=jax.ShapeDtypeStruct((NUM_RELS, DPAD), jnp.float32),
    mesh=plsc.VectorSubcoreMesh(
        core_axis_name="c", subcore_axis_name="s",
        num_cores=NC, num_subcores=NS),
    scratch_types=[
        pltpu.VMEM((B_PER_W,), jnp.int32),
        pltpu.VMEM((CHUNK, TPAD), jnp.float32),
        pltpu.VMEM((CHUNK, TPAD), jnp.float32),
        pltpu.SemaphoreType.DMA,
        pltpu.SemaphoreType.DMA,
        pltpu.SemaphoreType.DMA,
    ],
    compiler_params=pltpu.CompilerParams(
        needs_layout_passes=False, use_tc_tiling_on_sc=False),
)


@jax.jit
def kernel(obj_labels, rel_pair_idxs, prior_table):
    labels = obj_labels.astype(jnp.int32)
    pairs = rel_pair_idxs.astype(jnp.int32)
    heads = pairs[:, 0]
    tails = pairs[:, 1]
    table128 = lax.pad(
        prior_table, jnp.float32(0.0),
        ((0, 0, 0), (0, 0, 0), (0, TPAD - NUM_REL_CLS, 0)),
    ).reshape(NUM_OBJ_CLS * NUM_OBJ_CLS, TPAD)
    oh_flat, idx = _idx_oh(labels, heads, tails)
    rel64 = _rel_gather(idx, table128)
    return (oh_flat.reshape(NUM_OBJS, NUM_OBJ_CLS), rel64[:, :NUM_REL_CLS])
